# Initial kernel scaffold; baseline (speedup 1.0000x reference)
import jax
import jax.numpy as jnp
from jax.experimental import pallas as pl


def kernel(skills, W_enc, b_enc, codebook, W_dec, b_dec):
    raise NotImplementedError("write your pallas kernel here")


# speedup vs baseline: 2.4728x; 2.4728x over previous
import jax
import jax.numpy as jnp
from jax.experimental import pallas as pl
from jax.experimental.pallas import tpu as pltpu

EMBED = 1024
NT = 32
TD = 32
CB = 1024
BETA = 0.25
BB = 256


def _fused(skills_ref, W_enc_ref, b_enc_ref, cb_ref, W_dec_ref, b_dec_ref,
           recon_ref, codes_ref, s1_ref, s2_ref):
    x = skills_ref[...]
    p = jnp.dot(x, W_enc_ref[...], preferred_element_type=jnp.float32)
    p = p + b_enc_ref[...]

    C = cb_ref[...]
    cn = jnp.sum(C * C, axis=1)

    st_parts = []
    code_parts = []
    s1 = jnp.float32(0.0)
    for t in range(NT):
        pt = p[:, t * TD:(t + 1) * TD]
        fn = jnp.sum(pt * pt, axis=1, keepdims=True)
        s = jax.lax.dot_general(pt, C, (((1,), (1,)), ((), ())),
                                preferred_element_type=jnp.float32)
        d = fn - 2.0 * s + cn[None, :]
        m = jnp.min(d, axis=1, keepdims=True)
        iota = jax.lax.broadcasted_iota(jnp.int32, d.shape, 1)
        codes_t = jnp.min(jnp.where(d == m, iota, CB), axis=1, keepdims=True)
        onehot = (iota == codes_t).astype(jnp.float32)
        q = jnp.dot(onehot, C, preferred_element_type=jnp.float32)
        diff = pt - q
        s1 = s1 + jnp.sum(diff * diff)
        st_parts.append(pt + (q - pt))
        code_parts.append(codes_t)

    st = jnp.concatenate(st_parts, axis=1)
    codes = jnp.concatenate(code_parts, axis=1)
    recon = jnp.dot(st, W_dec_ref[...], preferred_element_type=jnp.float32)
    recon = recon + b_dec_ref[...]

    recon_ref[...] = recon
    codes_ref[...] = codes
    dr = recon - x
    s1_ref[...] = jnp.full((1, 1, 128), s1, jnp.float32)
    s2_ref[...] = jnp.full((1, 1, 128), jnp.sum(dr * dr), jnp.float32)


def kernel(skills, W_enc, b_enc, codebook, W_dec, b_dec):
    B = skills.shape[0]
    grid = B // BB
    b_enc2 = b_enc.reshape(1, EMBED)
    b_dec2 = b_dec.reshape(1, EMBED)
    recon, codes, s1, s2 = pl.pallas_call(
        _fused,
        grid=(grid,),
        in_specs=[
            pl.BlockSpec((BB, EMBED), lambda i: (i, 0)),
            pl.BlockSpec((EMBED, NT * TD), lambda i: (0, 0)),
            pl.BlockSpec((1, NT * TD), lambda i: (0, 0)),
            pl.BlockSpec((CB, TD), lambda i: (0, 0)),
            pl.BlockSpec((NT * TD, EMBED), lambda i: (0, 0)),
            pl.BlockSpec((1, EMBED), lambda i: (0, 0)),
        ],
        out_specs=[
            pl.BlockSpec((BB, EMBED), lambda i: (i, 0)),
            pl.BlockSpec((BB, NT), lambda i: (i, 0)),
            pl.BlockSpec((1, 1, 128), lambda i: (i, 0, 0)),
            pl.BlockSpec((1, 1, 128), lambda i: (i, 0, 0)),
        ],
        out_shape=(
            jax.ShapeDtypeStruct((B, EMBED), jnp.float32),
            jax.ShapeDtypeStruct((B, NT), jnp.int32),
            jax.ShapeDtypeStruct((grid, 1, 128), jnp.float32),
            jax.ShapeDtypeStruct((grid, 1, 128), jnp.float32),
        ),
        compiler_params=pltpu.CompilerParams(
            dimension_semantics=("parallel",),
        ),
    )(skills, W_enc, b_enc2, codebook, W_dec, b_dec2)

    denom = jnp.float32(B * EMBED)
    m = jnp.sum(s1[:, 0, 0]) / denom
    vq_loss = m + BETA * m
    loss = vq_loss + jnp.sum(s2[:, 0, 0]) / denom
    return recon, codes, loss


# speedup vs baseline: 3.0899x; 1.2496x over previous
import jax
import jax.numpy as jnp
from jax.experimental import pallas as pl
from jax.experimental.pallas import tpu as pltpu

EMBED = 1024
NT = 32
TD = 32
CB = 1024
BETA = 0.25
BB = 256


def _fused(skills_ref, W_enc_ref, b_enc_ref, cb_ref, W_dec_ref, b_dec_ref,
           recon_ref, codes_ref, s1_ref, s2_ref):
    x = skills_ref[...]
    p = jnp.dot(x, W_enc_ref[...], preferred_element_type=jnp.float32)
    p = p + b_enc_ref[...]

    C = cb_ref[...]
    cn = jnp.sum(C * C, axis=1)
    Cm2 = C * jnp.float32(-2.0)
    iota_f = jax.lax.broadcasted_iota(jnp.int32, (BB, CB), 1).astype(jnp.float32)

    st_parts = []
    code_parts = []
    s1 = jnp.float32(0.0)
    for t in range(NT):
        pt = p[:, t * TD:(t + 1) * TD]
        fn = jnp.sum(pt * pt, axis=1, keepdims=True)
        s = jax.lax.dot_general(pt, Cm2, (((1,), (1,)), ((), ())),
                                preferred_element_type=jnp.float32)
        d = (fn + s) + cn[None, :]
        m = jnp.min(d, axis=1, keepdims=True)
        eq = d == m
        codes_f = jnp.min(jnp.where(eq, iota_f, jnp.float32(CB)),
                          axis=1, keepdims=True)
        codes_t = codes_f.astype(jnp.int32)
        onehot = jnp.where(eq, jnp.float32(1.0), jnp.float32(0.0))
        q = jnp.dot(onehot, C, preferred_element_type=jnp.float32)
        diff = pt - q
        s1 = s1 + jnp.sum(diff * diff)
        st_parts.append(pt + (q - pt))
        code_parts.append(codes_t)

    st = jnp.concatenate(st_parts, axis=1)
    codes = jnp.concatenate(code_parts, axis=1)
    recon = jnp.dot(st, W_dec_ref[...], preferred_element_type=jnp.float32)
    recon = recon + b_dec_ref[...]

    recon_ref[...] = recon
    codes_ref[...] = codes
    dr = recon - x
    s1_ref[...] = jnp.full((1, 1, 128), s1, jnp.float32)
    s2_ref[...] = jnp.full((1, 1, 128), jnp.sum(dr * dr), jnp.float32)


def kernel(skills, W_enc, b_enc, codebook, W_dec, b_dec):
    B = skills.shape[0]
    grid = B // BB
    b_enc2 = b_enc.reshape(1, EMBED)
    b_dec2 = b_dec.reshape(1, EMBED)
    recon, codes, s1, s2 = pl.pallas_call(
        _fused,
        grid=(grid,),
        in_specs=[
            pl.BlockSpec((BB, EMBED), lambda i: (i, 0)),
            pl.BlockSpec((EMBED, NT * TD), lambda i: (0, 0)),
            pl.BlockSpec((1, NT * TD), lambda i: (0, 0)),
            pl.BlockSpec((CB, TD), lambda i: (0, 0)),
            pl.BlockSpec((NT * TD, EMBED), lambda i: (0, 0)),
            pl.BlockSpec((1, EMBED), lambda i: (0, 0)),
        ],
        out_specs=[
            pl.BlockSpec((BB, EMBED), lambda i: (i, 0)),
            pl.BlockSpec((BB, NT), lambda i: (i, 0)),
            pl.BlockSpec((1, 1, 128), lambda i: (i, 0, 0)),
            pl.BlockSpec((1, 1, 128), lambda i: (i, 0, 0)),
        ],
        out_shape=(
            jax.ShapeDtypeStruct((B, EMBED), jnp.float32),
            jax.ShapeDtypeStruct((B, NT), jnp.int32),
            jax.ShapeDtypeStruct((grid, 1, 128), jnp.float32),
            jax.ShapeDtypeStruct((grid, 1, 128), jnp.float32),
        ),
        compiler_params=pltpu.CompilerParams(
            dimension_semantics=("parallel",),
        ),
    )(skills, W_enc, b_enc2, codebook, W_dec, b_dec2)

    denom = jnp.float32(B * EMBED)
    m = jnp.sum(s1[:, 0, 0]) / denom
    vq_loss = m + BETA * m
    loss = vq_loss + jnp.sum(s2[:, 0, 0]) / denom
    return recon, codes, loss
